# initial kernel scaffold (unmeasured)
import jax
import jax.numpy as jnp
from jax import lax
from jax.experimental import pallas as pl
from jax.experimental.pallas import tpu as pltpu

NDEV = 32
M = 8192
KSH = 256
N = 4096
NH = N // 2
CH = M // NDEV

MESH = pl.DeviceIdType.MESH


def kernel(x, w_mat):
    def body(x_ref, w_ref, out_ref,
             acc_cw, acc_ccw, tmp_cw, tmp_ccw, comm_cw, comm_ccw,
             send_cw, recv_cw, send_ccw, recv_ccw,
             store_cw, store_ccw, store_acc,
             ack_cw, ack_ccw):
        my = lax.axis_index("i")
        left = (my - 1) % NDEV
        right = (my + 1) % NDEV

        bar = pltpu.get_barrier_semaphore()
        pl.semaphore_signal(bar, inc=1, device_id=(left,), device_id_type=MESH)
        pl.semaphore_signal(bar, inc=1, device_id=(right,), device_id_type=MESH)
        pl.semaphore_wait(bar, 2)

        def partial_cw(idx):
            return jnp.dot(
                x_ref[pl.ds(idx * CH, CH), :], w_ref[:, :NH],
                preferred_element_type=jnp.float32,
                precision=lax.Precision.HIGHEST,
            )

        def partial_ccw(idx):
            return jnp.dot(
                x_ref[pl.ds(idx * CH, CH), :], w_ref[:, NH:],
                preferred_element_type=jnp.float32,
                precision=lax.Precision.HIGHEST,
            )

        def signal_acks():
            pl.semaphore_signal(ack_cw, inc=1, device_id=(left,),
                                device_id_type=MESH)
            pl.semaphore_signal(ack_ccw, inc=1, device_id=(right,),
                                device_id_type=MESH)

        def wait_acks():
            pl.semaphore_wait(ack_cw, 1)
            pl.semaphore_wait(ack_ccw, 1)

        def start_ring_sends(src1, src2, slot):
            r1 = pltpu.make_async_remote_copy(
                src_ref=src1, dst_ref=comm_cw.at[slot],
                send_sem=send_cw.at[slot], recv_sem=recv_cw.at[slot],
                device_id=(right,), device_id_type=MESH)
            r2 = pltpu.make_async_remote_copy(
                src_ref=src2, dst_ref=comm_ccw.at[slot],
                send_sem=send_ccw.at[slot], recv_sem=recv_ccw.at[slot],
                device_id=(left,), device_id_type=MESH)
            r1.start()
            r2.start()
            return r1, r2

        def store_descs(t, slot):
            idx_cw = (my - t) % NDEV
            idx_ccw = (my + t) % NDEV
            s1 = pltpu.make_async_copy(
                comm_cw.at[slot],
                out_ref.at[pl.ds(idx_cw * CH, CH), pl.ds(0, NH)],
                store_cw.at[slot])
            s2 = pltpu.make_async_copy(
                comm_ccw.at[slot],
                out_ref.at[pl.ds(idx_ccw * CH, CH), pl.ds(NH, NH)],
                store_ccw.at[slot])
            return s1, s2

        acc_cw[...] = partial_cw(my)
        acc_ccw[...] = partial_ccw(my)

        def rs_step(k, slot, first):
            if not first:
                wait_acks()
            r1, r2 = start_ring_sends(acc_cw, acc_ccw, slot)
            tmp_cw[...] = partial_cw((my - k - 1) % NDEV)
            tmp_ccw[...] = partial_ccw((my + k + 1) % NDEV)
            r1.wait()
            r2.wait()
            acc_cw[...] = comm_cw[slot] + tmp_cw[...]
            acc_ccw[...] = comm_ccw[slot] + tmp_ccw[...]
            signal_acks()

        rs_step(0, 0, first=True)

        def rs_pair(j, carry):
            k = 2 * j + 1
            rs_step(k, 1, first=False)
            rs_step(k + 1, 0, first=False)
            return carry

        lax.fori_loop(0, (NDEV - 2) // 2, rs_pair, 0)

        st1 = pltpu.make_async_copy(
            acc_cw,
            out_ref.at[pl.ds(((my + 1) % NDEV) * CH, CH), pl.ds(0, NH)],
            store_acc.at[0])
        st2 = pltpu.make_async_copy(
            acc_ccw,
            out_ref.at[pl.ds(((my - 1) % NDEV) * CH, CH), pl.ds(NH, NH)],
            store_acc.at[1])
        st1.start()
        st2.start()

        def ag_step(t, slot, prev_slot, first):
            wait_acks()
            if first:
                r1, r2 = start_ring_sends(acc_cw, acc_ccw, slot)
            else:
                r1, r2 = start_ring_sends(
                    comm_cw.at[prev_slot], comm_ccw.at[prev_slot], slot)
            r1.wait()
            r2.wait()
            if not first:
                p1, p2 = store_descs(t - 1, prev_slot)
                p1.wait()
                p2.wait()
            s1, s2 = store_descs(t, slot)
            s1.start()
            s2.start()
            signal_acks()

        ag_step(0, 1, 0, first=True)

        def ag_pair(j, carry):
            t = 2 * j + 1
            ag_step(t, 0, 1, first=False)
            ag_step(t + 1, 1, 0, first=False)
            return carry

        lax.fori_loop(0, (NDEV - 2) // 2, ag_pair, 0)

        f1, f2 = store_descs(NDEV - 2, 1)
        f1.wait()
        f2.wait()
        st1.wait()
        st2.wait()
        wait_acks()

    try:
        params = pltpu.CompilerParams(collective_id=0)
    except AttributeError:
        params = pltpu.TPUCompilerParams(collective_id=0)

    return pl.pallas_call(
        body,
        out_shape=jax.ShapeDtypeStruct((M, N), jnp.float32),
        in_specs=[
            pl.BlockSpec(memory_space=pltpu.VMEM),
            pl.BlockSpec(memory_space=pltpu.VMEM),
        ],
        out_specs=pl.BlockSpec(memory_space=pltpu.ANY),
        scratch_shapes=[
            pltpu.VMEM((CH, NH), jnp.float32),
            pltpu.VMEM((CH, NH), jnp.float32),
            pltpu.VMEM((CH, NH), jnp.float32),
            pltpu.VMEM((CH, NH), jnp.float32),
            pltpu.VMEM((2, CH, NH), jnp.float32),
            pltpu.VMEM((2, CH, NH), jnp.float32),
            pltpu.SemaphoreType.DMA((2,)),
            pltpu.SemaphoreType.DMA((2,)),
            pltpu.SemaphoreType.DMA((2,)),
            pltpu.SemaphoreType.DMA((2,)),
            pltpu.SemaphoreType.DMA((2,)),
            pltpu.SemaphoreType.DMA((2,)),
            pltpu.SemaphoreType.DMA((2,)),
            pltpu.SemaphoreType.REGULAR,
            pltpu.SemaphoreType.REGULAR,
        ],
        compiler_params=params,
    )(x, w_mat)


# baseline (device time: 3050493 ns/iter reference)
import jax
import jax.numpy as jnp
from jax import lax
from jax.experimental import pallas as pl
from jax.experimental.pallas import tpu as pltpu

NDEV = 32
M = 8192
KSH = 256
N = 4096
NH = N // 2
CH = M // NDEV

MESH = pl.DeviceIdType.MESH


def kernel(x, w_mat):
    def body(x_ref, w_ref, out_ref,
             acc_cw, acc_ccw, tmp_cw, tmp_ccw, comm_cw, comm_ccw,
             send_cw, recv_cw, send_ccw, recv_ccw,
             store_cw, store_ccw, store_acc,
             ack_cw, ack_ccw):
        my = lax.axis_index("i")
        left = (my - 1) % NDEV
        right = (my + 1) % NDEV

        bar = pltpu.get_barrier_semaphore()
        pl.semaphore_signal(bar, inc=1, device_id=(left,), device_id_type=MESH)
        pl.semaphore_signal(bar, inc=1, device_id=(right,), device_id_type=MESH)
        pl.semaphore_wait(bar, 2)

        def partial_cw(idx):
            return jnp.dot(
                x_ref[pl.ds(idx * CH, CH), :], w_ref[:, :NH],
                preferred_element_type=jnp.float32,
                precision=lax.Precision.HIGHEST,
            )

        def partial_ccw(idx):
            return jnp.dot(
                x_ref[pl.ds(idx * CH, CH), :], w_ref[:, NH:],
                preferred_element_type=jnp.float32,
                precision=lax.Precision.HIGHEST,
            )

        def signal_acks():
            pl.semaphore_signal(ack_cw, inc=1, device_id=(left,),
                                device_id_type=MESH)
            pl.semaphore_signal(ack_ccw, inc=1, device_id=(right,),
                                device_id_type=MESH)

        def wait_acks():
            pl.semaphore_wait(ack_cw, 1)
            pl.semaphore_wait(ack_ccw, 1)

        def start_ring_sends(src1, src2, slot):
            r1 = pltpu.make_async_remote_copy(
                src_ref=src1, dst_ref=comm_cw.at[slot],
                send_sem=send_cw.at[slot], recv_sem=recv_cw.at[slot],
                device_id=(right,), device_id_type=MESH)
            r2 = pltpu.make_async_remote_copy(
                src_ref=src2, dst_ref=comm_ccw.at[slot],
                send_sem=send_ccw.at[slot], recv_sem=recv_ccw.at[slot],
                device_id=(left,), device_id_type=MESH)
            r1.start()
            r2.start()
            return r1, r2

        def store_descs(t, slot):
            idx_cw = (my - t) % NDEV
            idx_ccw = (my + t) % NDEV
            s1 = pltpu.make_async_copy(
                comm_cw.at[slot],
                out_ref.at[pl.ds(idx_cw * CH, CH), pl.ds(0, NH)],
                store_cw.at[slot])
            s2 = pltpu.make_async_copy(
                comm_ccw.at[slot],
                out_ref.at[pl.ds(idx_ccw * CH, CH), pl.ds(NH, NH)],
                store_ccw.at[slot])
            return s1, s2

        acc_cw[...] = partial_cw(my)
        acc_ccw[...] = partial_ccw(my)

        def rs_step(k, slot, first):
            if not first:
                wait_acks()
            r1, r2 = start_ring_sends(acc_cw, acc_ccw, slot)
            tmp_cw[...] = partial_cw((my - k - 1) % NDEV)
            tmp_ccw[...] = partial_ccw((my + k + 1) % NDEV)
            r1.wait()
            r2.wait()
            acc_cw[...] = comm_cw[slot] + tmp_cw[...]
            acc_ccw[...] = comm_ccw[slot] + tmp_ccw[...]
            signal_acks()

        rs_step(0, 0, first=True)

        def rs_pair(j, carry):
            k = 2 * j + 1
            rs_step(k, 1, first=False)
            rs_step(k + 1, 0, first=False)
            return carry

        lax.fori_loop(0, (NDEV - 2) // 2, rs_pair, 0)

        st1 = pltpu.make_async_copy(
            acc_cw,
            out_ref.at[pl.ds(((my + 1) % NDEV) * CH, CH), pl.ds(0, NH)],
            store_acc.at[0])
        st2 = pltpu.make_async_copy(
            acc_ccw,
            out_ref.at[pl.ds(((my - 1) % NDEV) * CH, CH), pl.ds(NH, NH)],
            store_acc.at[1])
        st1.start()
        st2.start()

        def ag_step(t, slot, prev_slot, first):
            wait_acks()
            if first:
                r1, r2 = start_ring_sends(acc_cw, acc_ccw, slot)
            else:
                r1, r2 = start_ring_sends(
                    comm_cw.at[prev_slot], comm_ccw.at[prev_slot], slot)
            r1.wait()
            r2.wait()
            if not first:
                p1, p2 = store_descs(t - 1, prev_slot)
                p1.wait()
                p2.wait()
            s1, s2 = store_descs(t, slot)
            s1.start()
            s2.start()
            signal_acks()

        ag_step(0, 1, 0, first=True)

        def ag_pair(j, carry):
            t = 2 * j + 1
            ag_step(t, 0, 1, first=False)
            ag_step(t + 1, 1, 0, first=False)
            return carry

        lax.fori_loop(0, (NDEV - 2) // 2, ag_pair, 0)

        f1, f2 = store_descs(NDEV - 2, 1)
        f1.wait()
        f2.wait()
        st1.wait()
        st2.wait()
        wait_acks()

    try:
        params = pltpu.CompilerParams(collective_id=0)
    except AttributeError:
        params = pltpu.TPUCompilerParams(collective_id=0)

    return pl.pallas_call(
        body,
        out_shape=jax.ShapeDtypeStruct((M, N), jnp.float32),
        in_specs=[
            pl.BlockSpec(memory_space=pltpu.VMEM),
            pl.BlockSpec(memory_space=pltpu.VMEM),
        ],
        out_specs=pl.BlockSpec(memory_space=pl.ANY),
        scratch_shapes=[
            pltpu.VMEM((CH, NH), jnp.float32),
            pltpu.VMEM((CH, NH), jnp.float32),
            pltpu.VMEM((CH, NH), jnp.float32),
            pltpu.VMEM((CH, NH), jnp.float32),
            pltpu.VMEM((2, CH, NH), jnp.float32),
            pltpu.VMEM((2, CH, NH), jnp.float32),
            pltpu.SemaphoreType.DMA((2,)),
            pltpu.SemaphoreType.DMA((2,)),
            pltpu.SemaphoreType.DMA((2,)),
            pltpu.SemaphoreType.DMA((2,)),
            pltpu.SemaphoreType.DMA((2,)),
            pltpu.SemaphoreType.DMA((2,)),
            pltpu.SemaphoreType.DMA((2,)),
            pltpu.SemaphoreType.REGULAR,
            pltpu.SemaphoreType.REGULAR,
        ],
        compiler_params=params,
    )(x, w_mat)


# device time: 1666231 ns/iter; 1.8308x vs baseline; 1.8308x over previous
import jax
import jax.numpy as jnp
from jax import lax
from jax.experimental import pallas as pl
from jax.experimental.pallas import tpu as pltpu

NDEV = 32
M = 8192
KSH = 256
N = 4096
NH = N // 2
CH = M // NDEV

MESH = pl.DeviceIdType.MESH

_PLANE_ORDER = [(0, 0), (1, 0), (1, 1), (0, 1),
                (0, 2), (1, 2), (1, 3), (0, 3)]
_LOGICAL_OF_COORD = {}
for _z in range(4):
    for _x, _y in _PLANE_ORDER:
        _LOGICAL_OF_COORD[(_x, _y, _z)] = len(_LOGICAL_OF_COORD)

_PATH_YZ = [(0, 0), (1, 0), (2, 0), (3, 0), (3, 1), (2, 1), (1, 1), (0, 1),
            (0, 2), (1, 2), (2, 2), (3, 2), (3, 3), (2, 3), (1, 3), (0, 3)]
_RING_COORDS = ([(0, y, z) for y, z in _PATH_YZ]
                + [(1, y, z) for y, z in reversed(_PATH_YZ)])
RING_LOGICAL = [_LOGICAL_OF_COORD[c] for c in _RING_COORDS]
POS_OF_LOGICAL = [0] * NDEV
for _p, _l in enumerate(RING_LOGICAL):
    POS_OF_LOGICAL[_l] = _p


def kernel(x, w_mat):
    def body(scal_ref, x_ref, w_ref, out_ref,
             acc_cw, acc_ccw, tmp_cw, tmp_ccw, comm_cw, comm_ccw,
             send_cw, recv_cw, send_ccw, recv_ccw,
             store_cw, store_ccw, store_acc,
             ack_cw, ack_ccw):
        my = scal_ref[0]
        left = scal_ref[1]
        right = scal_ref[2]

        bar = pltpu.get_barrier_semaphore()
        pl.semaphore_signal(bar, inc=1, device_id=(left,), device_id_type=MESH)
        pl.semaphore_signal(bar, inc=1, device_id=(right,), device_id_type=MESH)
        pl.semaphore_wait(bar, 2)

        def partial_cw(idx):
            return jnp.dot(
                x_ref[pl.ds(idx * CH, CH), :], w_ref[:, :NH],
                preferred_element_type=jnp.float32,
                precision=lax.Precision.HIGHEST,
            )

        def partial_ccw(idx):
            return jnp.dot(
                x_ref[pl.ds(idx * CH, CH), :], w_ref[:, NH:],
                preferred_element_type=jnp.float32,
                precision=lax.Precision.HIGHEST,
            )

        def signal_acks():
            pl.semaphore_signal(ack_cw, inc=1, device_id=(left,),
                                device_id_type=MESH)
            pl.semaphore_signal(ack_ccw, inc=1, device_id=(right,),
                                device_id_type=MESH)

        def wait_acks():
            pl.semaphore_wait(ack_cw, 1)
            pl.semaphore_wait(ack_ccw, 1)

        def start_ring_sends(src1, src2, slot):
            r1 = pltpu.make_async_remote_copy(
                src_ref=src1, dst_ref=comm_cw.at[slot],
                send_sem=send_cw.at[slot], recv_sem=recv_cw.at[slot],
                device_id=(right,), device_id_type=MESH)
            r2 = pltpu.make_async_remote_copy(
                src_ref=src2, dst_ref=comm_ccw.at[slot],
                send_sem=send_ccw.at[slot], recv_sem=recv_ccw.at[slot],
                device_id=(left,), device_id_type=MESH)
            r1.start()
            r2.start()
            return r1, r2

        def store_descs(t, slot):
            idx_cw = (my - t) % NDEV
            idx_ccw = (my + t) % NDEV
            s1 = pltpu.make_async_copy(
                comm_cw.at[slot],
                out_ref.at[pl.ds(idx_cw * CH, CH), pl.ds(0, NH)],
                store_cw.at[slot])
            s2 = pltpu.make_async_copy(
                comm_ccw.at[slot],
                out_ref.at[pl.ds(idx_ccw * CH, CH), pl.ds(NH, NH)],
                store_ccw.at[slot])
            return s1, s2

        acc_cw[...] = partial_cw(my)
        acc_ccw[...] = partial_ccw(my)

        def rs_step(k, slot, first):
            if not first:
                wait_acks()
            r1, r2 = start_ring_sends(acc_cw, acc_ccw, slot)
            tmp_cw[...] = partial_cw((my - k - 1) % NDEV)
            tmp_ccw[...] = partial_ccw((my + k + 1) % NDEV)
            r1.wait()
            r2.wait()
            acc_cw[...] = comm_cw[slot] + tmp_cw[...]
            acc_ccw[...] = comm_ccw[slot] + tmp_ccw[...]
            signal_acks()

        rs_step(0, 0, first=True)

        def rs_pair(j, carry):
            k = 2 * j + 1
            rs_step(k, 1, first=False)
            rs_step(k + 1, 0, first=False)
            return carry

        lax.fori_loop(0, (NDEV - 2) // 2, rs_pair, 0)

        st1 = pltpu.make_async_copy(
            acc_cw,
            out_ref.at[pl.ds(((my + 1) % NDEV) * CH, CH), pl.ds(0, NH)],
            store_acc.at[0])
        st2 = pltpu.make_async_copy(
            acc_ccw,
            out_ref.at[pl.ds(((my - 1) % NDEV) * CH, CH), pl.ds(NH, NH)],
            store_acc.at[1])
        st1.start()
        st2.start()

        def ag_step(t, slot, prev_slot, first):
            wait_acks()
            if first:
                r1, r2 = start_ring_sends(acc_cw, acc_ccw, slot)
            else:
                r1, r2 = start_ring_sends(
                    comm_cw.at[prev_slot], comm_ccw.at[prev_slot], slot)
            r1.wait()
            r2.wait()
            if not first:
                p1, p2 = store_descs(t - 1, prev_slot)
                p1.wait()
                p2.wait()
            s1, s2 = store_descs(t, slot)
            s1.start()
            s2.start()
            signal_acks()

        ag_step(0, 1, 0, first=True)

        def ag_pair(j, carry):
            t = 2 * j + 1
            ag_step(t, 0, 1, first=False)
            ag_step(t + 1, 1, 0, first=False)
            return carry

        lax.fori_loop(0, (NDEV - 2) // 2, ag_pair, 0)

        f1, f2 = store_descs(NDEV - 2, 1)
        f1.wait()
        f2.wait()
        st1.wait()
        st2.wait()
        wait_acks()

    try:
        params = pltpu.CompilerParams(collective_id=0)
    except AttributeError:
        params = pltpu.TPUCompilerParams(collective_id=0)

    i = lax.axis_index("i")
    pos_t = jnp.asarray(POS_OF_LOGICAL, dtype=jnp.int32)
    ring_t = jnp.asarray(RING_LOGICAL, dtype=jnp.int32)
    r = pos_t[i]
    scalars = jnp.stack([
        r,
        ring_t[(r - 1) % NDEV],
        ring_t[(r + 1) % NDEV],
    ]).astype(jnp.int32)

    return pl.pallas_call(
        body,
        out_shape=jax.ShapeDtypeStruct((M, N), jnp.float32),
        in_specs=[
            pl.BlockSpec(memory_space=pltpu.MemorySpace.SMEM),
            pl.BlockSpec(memory_space=pltpu.VMEM),
            pl.BlockSpec(memory_space=pltpu.VMEM),
        ],
        out_specs=pl.BlockSpec(memory_space=pl.ANY),
        scratch_shapes=[
            pltpu.VMEM((CH, NH), jnp.float32),
            pltpu.VMEM((CH, NH), jnp.float32),
            pltpu.VMEM((CH, NH), jnp.float32),
            pltpu.VMEM((CH, NH), jnp.float32),
            pltpu.VMEM((2, CH, NH), jnp.float32),
            pltpu.VMEM((2, CH, NH), jnp.float32),
            pltpu.SemaphoreType.DMA((2,)),
            pltpu.SemaphoreType.DMA((2,)),
            pltpu.SemaphoreType.DMA((2,)),
            pltpu.SemaphoreType.DMA((2,)),
            pltpu.SemaphoreType.DMA((2,)),
            pltpu.SemaphoreType.DMA((2,)),
            pltpu.SemaphoreType.DMA((2,)),
            pltpu.SemaphoreType.REGULAR,
            pltpu.SemaphoreType.REGULAR,
        ],
        compiler_params=params,
    )(scalars, x, w_mat)
